# baseline (speedup 1.0000x reference)
import functools
import jax
import jax.numpy as jnp
from jax import lax
from jax.experimental import pallas as pl
from jax.experimental.pallas import tpu as pltpu
from jax.experimental.pallas import tpu_sc as plsc

_NW = 32
_CH = 128


def _gather_body(nch, per_w, table_hbm, idx_hbm, out_hbm, idx_v, rows_v, sem):
    import numpy as np
    two = np.int32(2)
    wid = lax.axis_index("s") * two + lax.axis_index("c")
    base = wid * np.int32(per_w)
    pltpu.sync_copy(idx_hbm.at[pl.ds(base, per_w)], idx_v)

    for i in range(nch):
        off = np.int32(i * _CH)
        pltpu.async_copy(table_hbm.at[idx_v.at[pl.ds(off, _CH)]], rows_v, sem).wait()
        pltpu.sync_copy(rows_v, out_hbm.at[pl.ds(base + off, _CH)])


def _gather_rows(table, idx, dtype):
    b = idx.shape[0]
    d = table.shape[1]
    per_w = b // _NW
    nch = per_w // _CH
    mesh = plsc.VectorSubcoreMesh(core_axis_name="c", subcore_axis_name="s")
    kf = functools.partial(
        pl.kernel,
        mesh=mesh,
        out_type=jax.ShapeDtypeStruct((b, d), dtype),
        scratch_types=[
            pltpu.VMEM((per_w,), jnp.int32),
            pltpu.VMEM((_CH, d), dtype),
            pltpu.SemaphoreType.DMA,
        ],
    )(functools.partial(_gather_body, nch, per_w))
    return kf(table, idx)

_TR = 2000


import numpy as np

_I0 = np.int32(0)


def _stage1_body(xn_ref, w1_ref, b1_ref, h_ref, s1_ref, s2_ref):
    h = jnp.dot(xn_ref[...], w1_ref[...], preferred_element_type=jnp.float32)
    h = h + b1_ref[...]
    h_ref[...] = h
    s1_ref[...] = jnp.sum(h, axis=0, keepdims=True)[None]
    s2_ref[...] = jnp.sum(h * h, axis=0, keepdims=True)[None]


def _node_stage1(xn, w1, b1):
    n = xn.shape[0]
    g = n // _TR
    kf = pl.pallas_call(
        _stage1_body,
        grid=(g,),
        in_specs=[
            pl.BlockSpec((_TR, xn.shape[1]), lambda i: (i, _I0)),
            pl.BlockSpec((xn.shape[1], 64), lambda i: (_I0, _I0)),
            pl.BlockSpec((1, 64), lambda i: (_I0, _I0)),
        ],
        out_specs=[
            pl.BlockSpec((_TR, 64), lambda i: (i, _I0)),
            pl.BlockSpec((1, 1, 64), lambda i: (i, _I0, _I0)),
            pl.BlockSpec((1, 1, 64), lambda i: (i, _I0, _I0)),
        ],
        out_shape=[
            jax.ShapeDtypeStruct((n, 64), jnp.float32),
            jax.ShapeDtypeStruct((g, 1, 64), jnp.float32),
            jax.ShapeDtypeStruct((g, 1, 64), jnp.float32),
        ],
    )
    return kf(xn, w1, b1.reshape(1, 64))


def _stage2_body(h_ref, sc_ref, tt_ref, w2_ref, b2_ref, wa_ref, x_ref, a_ref):
    hb = h_ref[...] * sc_ref[...] + tt_ref[...]
    x = jnp.dot(jnp.maximum(hb, 0.0), w2_ref[...],
                preferred_element_type=jnp.float32) + b2_ref[...]
    x_ref[...] = x
    a_ref[...] = jnp.dot(x, wa_ref[...], preferred_element_type=jnp.float32)


def _node_stage2(h, sc, tt, w2, b2, wa):
    n = h.shape[0]
    g = n // _TR
    kf = pl.pallas_call(
        _stage2_body,
        grid=(g,),
        in_specs=[
            pl.BlockSpec((_TR, 64), lambda i: (i, _I0)),
            pl.BlockSpec((1, 64), lambda i: (_I0, _I0)),
            pl.BlockSpec((1, 64), lambda i: (_I0, _I0)),
            pl.BlockSpec((64, 64), lambda i: (_I0, _I0)),
            pl.BlockSpec((1, 64), lambda i: (_I0, _I0)),
            pl.BlockSpec((64, 128), lambda i: (_I0, _I0)),
        ],
        out_specs=[
            pl.BlockSpec((_TR, 64), lambda i: (i, _I0)),
            pl.BlockSpec((_TR, 128), lambda i: (i, _I0)),
        ],
        out_shape=[
            jax.ShapeDtypeStruct((n, 64), jnp.float32),
            jax.ShapeDtypeStruct((n, 128), jnp.float32),
        ],
    )
    return kf(h, sc.reshape(1, 64), tt.reshape(1, 64), w2, b2.reshape(1, 64), wa)


def kernel(path, free, collided, obstacles, edge_index, loop, node_w1, node_b1,
           bn_g, bn_b, node_w2, node_b2, l0a_w, l0a_b, l0b_w, l0b_b,
           l1a_w, l1a_b, l1b_w, l1b_b, sm_w, sm_b):
    P = path.shape[0]
    Fn = free.shape[0]
    C = collided.shape[0]
    N = P + Fn + C

    path = path.astype(jnp.float32)
    cand = jnp.concatenate([free, collided], axis=0).astype(jnp.float32)

    src0 = edge_index[0]
    dst0 = edge_index[1]
    sent64 = jnp.int64(2**31 - 1)
    keys = jnp.where(dst0 < P, dst0.astype(jnp.int64) * N + src0.astype(jnp.int64), sent64)
    ks = jnp.sort(keys.astype(jnp.int32))
    sent = jnp.int32(2**31 - 1)
    m_rel = jnp.sum((ks < sent).astype(jnp.int32))
    cap = 32768

    r = jnp.arange(N, dtype=jnp.int32)
    info = jnp.stack([(r < P).astype(jnp.float32),
                      ((r >= P) & (r < P + Fn)).astype(jnp.float32),
                      (r >= P + Fn).astype(jnp.float32)], axis=1)

    w0 = l0a_w[:64]
    w1m = l0a_w[64:128]
    w2m = l0a_w[128:]
    wa = jnp.concatenate([w0 + w1m, jnp.zeros((64, 64), jnp.float32)],
                         axis=1)
    wb = w2m - w0

    c2 = jnp.sum(cand * cand, axis=1)

    for _ in range(2):
        nodes = jnp.concatenate([path, cand], axis=0)
        xn = jnp.concatenate([nodes, info], axis=1)
        h, s1, s2 = _node_stage1(xn, node_w1, node_b1)
        mu = jnp.sum(s1, axis=(0, 1)) / N
        var = jnp.sum(s2, axis=(0, 1)) / N - mu * mu
        sc = bn_g / jnp.sqrt(var + 1e-5)
        tt = bn_b - mu * sc
        x, a_tab = _node_stage2(h, sc, tt, node_w2, node_b2, wa)
        xp = x[:P]
        bb = xp @ wb + l0a_b

        p2 = jnp.sum(path * path, axis=1)
        d2 = p2[:, None] + c2[None, :] - 2.0 * (path @ cand.T)
        _, nn = lax.top_k(-d2, 10)

        nn16 = jnp.concatenate(
            [nn, jnp.full((P, 6), -1, jnp.int32)], axis=1)
        comb = jnp.concatenate(
            [bb, lax.bitcast_convert_type(nn16, jnp.float32),
             jnp.zeros((P, 48), jnp.float32)], axis=1)

        def base_agg(ks_slice, use_sc):
            valid = ks_slice < sent
            first = jnp.concatenate(
                [valid[:1], (ks_slice[1:] != ks_slice[:-1]) & valid[1:]])
            d_e = jnp.where(valid, ks_slice // N, 0)
            s_e = jnp.where(valid, ks_slice % N, 0)
            if use_sc:
                rows_a = _gather_rows(a_tab, s_e, jnp.float32)[:, :64]
                g_c = _gather_rows(comb, d_e, jnp.float32)
                rows_b = g_c[:, :64]
                nnrows = lax.bitcast_convert_type(g_c[:, 64:80], jnp.int32)
            else:
                nnrows = nn16[d_e]
                rows_a = a_tab[s_e, :64]
                rows_b = bb[d_e]
            match = jnp.any(nnrows == (s_e - P)[:, None], axis=1) & (s_e >= P)
            wgt = (first & ~match).astype(jnp.float32)
            relu_e = jnp.maximum(rows_a + rows_b, 0.0) * wgt[:, None]
            aggu = jax.ops.segment_sum(relu_e, d_e, num_segments=P)
            cnt = jax.ops.segment_sum(wgt, d_e, num_segments=P)
            return aggu, cnt

        aggu, cnt = lax.cond(m_rel <= cap,
                             lambda: base_agg(ks[:cap], True),
                             lambda: base_agg(ks, False))
        cnt = cnt + 10.0

        relu_n = jnp.maximum(a_tab[nn + P, :64] + bb[:, None, :], 0.0)
        aggu = aggu + jnp.sum(relu_n, axis=1)

        agg = aggu @ l0b_w + cnt[:, None] * l0b_b
        hnp = xp + jnp.maximum(agg @ l1a_w + l1a_b, 0.0) @ l1b_w + l1b_b
        sm = hnp @ sm_w + sm_b
        path = path.at[1:-1].set(sm[1:-1])

    return path


# speedup vs baseline: 1.0002x; 1.0002x over previous
import functools
import jax
import jax.numpy as jnp
from jax import lax
from jax.experimental import pallas as pl
from jax.experimental.pallas import tpu as pltpu
from jax.experimental.pallas import tpu_sc as plsc

_NW = 32
_CH = 128


def _gather_body(nch, per_w, table_hbm, idx_hbm, out_hbm, idx_v, rows_v, sem):
    import numpy as np
    two = np.int32(2)
    wid = lax.axis_index("s") * two + lax.axis_index("c")
    base = wid * np.int32(per_w)
    pltpu.sync_copy(idx_hbm.at[pl.ds(base, per_w)], idx_v)

    nb = 4
    for g0 in range(0, nch, nb):
        k = min(nb, nch - g0)
        cps = []
        for j in range(k):
            off = np.int32((g0 + j) * _CH)
            cps.append(pltpu.async_copy(
                table_hbm.at[idx_v.at[pl.ds(off, _CH)]], rows_v.at[np.int32(j)], sem))
        for j in range(k):
            cps[j].wait()
        for j in range(k):
            off = np.int32((g0 + j) * _CH)
            pltpu.sync_copy(rows_v.at[np.int32(j)], out_hbm.at[pl.ds(base + off, _CH)])


def _gather_rows(table, idx, dtype):
    b = idx.shape[0]
    d = table.shape[1]
    per_w = b // _NW
    nch = per_w // _CH
    mesh = plsc.VectorSubcoreMesh(core_axis_name="c", subcore_axis_name="s")
    kf = functools.partial(
        pl.kernel,
        mesh=mesh,
        out_type=jax.ShapeDtypeStruct((b, d), dtype),
        scratch_types=[
            pltpu.VMEM((per_w,), jnp.int32),
            pltpu.VMEM((4, _CH, d), dtype),
            pltpu.SemaphoreType.DMA,
        ],
    )(functools.partial(_gather_body, nch, per_w))
    return kf(table, idx)

_TR = 2000


import numpy as np

_I0 = np.int32(0)


def _stage1_body(xn_ref, w1_ref, b1_ref, h_ref, s1_ref, s2_ref):
    h = jnp.dot(xn_ref[...], w1_ref[...], preferred_element_type=jnp.float32)
    h = h + b1_ref[...]
    h_ref[...] = h
    s1_ref[...] = jnp.sum(h, axis=0, keepdims=True)[None]
    s2_ref[...] = jnp.sum(h * h, axis=0, keepdims=True)[None]


def _node_stage1(xn, w1, b1):
    n = xn.shape[0]
    g = n // _TR
    kf = pl.pallas_call(
        _stage1_body,
        grid=(g,),
        in_specs=[
            pl.BlockSpec((_TR, xn.shape[1]), lambda i: (i, _I0)),
            pl.BlockSpec((xn.shape[1], 64), lambda i: (_I0, _I0)),
            pl.BlockSpec((1, 64), lambda i: (_I0, _I0)),
        ],
        out_specs=[
            pl.BlockSpec((_TR, 64), lambda i: (i, _I0)),
            pl.BlockSpec((1, 1, 64), lambda i: (i, _I0, _I0)),
            pl.BlockSpec((1, 1, 64), lambda i: (i, _I0, _I0)),
        ],
        out_shape=[
            jax.ShapeDtypeStruct((n, 64), jnp.float32),
            jax.ShapeDtypeStruct((g, 1, 64), jnp.float32),
            jax.ShapeDtypeStruct((g, 1, 64), jnp.float32),
        ],
    )
    return kf(xn, w1, b1.reshape(1, 64))


def _stage2_body(h_ref, sc_ref, tt_ref, w2_ref, b2_ref, wa_ref, x_ref, a_ref):
    hb = h_ref[...] * sc_ref[...] + tt_ref[...]
    x = jnp.dot(jnp.maximum(hb, 0.0), w2_ref[...],
                preferred_element_type=jnp.float32) + b2_ref[...]
    x_ref[...] = x
    a_ref[...] = jnp.dot(x, wa_ref[...], preferred_element_type=jnp.float32)


def _node_stage2(h, sc, tt, w2, b2, wa):
    n = h.shape[0]
    g = n // _TR
    kf = pl.pallas_call(
        _stage2_body,
        grid=(g,),
        in_specs=[
            pl.BlockSpec((_TR, 64), lambda i: (i, _I0)),
            pl.BlockSpec((1, 64), lambda i: (_I0, _I0)),
            pl.BlockSpec((1, 64), lambda i: (_I0, _I0)),
            pl.BlockSpec((64, 64), lambda i: (_I0, _I0)),
            pl.BlockSpec((1, 64), lambda i: (_I0, _I0)),
            pl.BlockSpec((64, 128), lambda i: (_I0, _I0)),
        ],
        out_specs=[
            pl.BlockSpec((_TR, 64), lambda i: (i, _I0)),
            pl.BlockSpec((_TR, 128), lambda i: (i, _I0)),
        ],
        out_shape=[
            jax.ShapeDtypeStruct((n, 64), jnp.float32),
            jax.ShapeDtypeStruct((n, 128), jnp.float32),
        ],
    )
    return kf(h, sc.reshape(1, 64), tt.reshape(1, 64), w2, b2.reshape(1, 64), wa)


def kernel(path, free, collided, obstacles, edge_index, loop, node_w1, node_b1,
           bn_g, bn_b, node_w2, node_b2, l0a_w, l0a_b, l0b_w, l0b_b,
           l1a_w, l1a_b, l1b_w, l1b_b, sm_w, sm_b):
    P = path.shape[0]
    Fn = free.shape[0]
    C = collided.shape[0]
    N = P + Fn + C

    path = path.astype(jnp.float32)
    cand = jnp.concatenate([free, collided], axis=0).astype(jnp.float32)

    src0 = edge_index[0]
    dst0 = edge_index[1]
    sent64 = jnp.int64(2**31 - 1)
    keys = jnp.where(dst0 < P, dst0.astype(jnp.int64) * N + src0.astype(jnp.int64), sent64)
    ks = jnp.sort(keys.astype(jnp.int32))
    sent = jnp.int32(2**31 - 1)
    m_rel = jnp.sum((ks < sent).astype(jnp.int32))
    cap = 32768

    r = jnp.arange(N, dtype=jnp.int32)
    info = jnp.stack([(r < P).astype(jnp.float32),
                      ((r >= P) & (r < P + Fn)).astype(jnp.float32),
                      (r >= P + Fn).astype(jnp.float32)], axis=1)

    w0 = l0a_w[:64]
    w1m = l0a_w[64:128]
    w2m = l0a_w[128:]
    wa = jnp.concatenate([w0 + w1m, jnp.zeros((64, 64), jnp.float32)],
                         axis=1)
    wb = w2m - w0

    c2 = jnp.sum(cand * cand, axis=1)

    for _ in range(2):
        nodes = jnp.concatenate([path, cand], axis=0)
        xn = jnp.concatenate([nodes, info], axis=1)
        h, s1, s2 = _node_stage1(xn, node_w1, node_b1)
        mu = jnp.sum(s1, axis=(0, 1)) / N
        var = jnp.sum(s2, axis=(0, 1)) / N - mu * mu
        sc = bn_g / jnp.sqrt(var + 1e-5)
        tt = bn_b - mu * sc
        x, a_tab = _node_stage2(h, sc, tt, node_w2, node_b2, wa)
        xp = x[:P]
        bb = xp @ wb + l0a_b

        p2 = jnp.sum(path * path, axis=1)
        d2 = p2[:, None] + c2[None, :] - 2.0 * (path @ cand.T)
        _, nn = lax.top_k(-d2, 10)

        nn16 = jnp.concatenate(
            [nn, jnp.full((P, 6), -1, jnp.int32)], axis=1)
        comb = jnp.concatenate(
            [bb, lax.bitcast_convert_type(nn16, jnp.float32),
             jnp.zeros((P, 48), jnp.float32)], axis=1)

        def base_agg(ks_slice, use_sc):
            valid = ks_slice < sent
            first = jnp.concatenate(
                [valid[:1], (ks_slice[1:] != ks_slice[:-1]) & valid[1:]])
            d_e = jnp.where(valid, ks_slice // N, 0)
            s_e = jnp.where(valid, ks_slice % N, 0)
            if use_sc:
                rows_a = _gather_rows(a_tab, s_e, jnp.float32)[:, :64]
                g_c = _gather_rows(comb, d_e, jnp.float32)
                rows_b = g_c[:, :64]
                nnrows = lax.bitcast_convert_type(g_c[:, 64:80], jnp.int32)
            else:
                nnrows = nn16[d_e]
                rows_a = a_tab[s_e, :64]
                rows_b = bb[d_e]
            match = jnp.any(nnrows == (s_e - P)[:, None], axis=1) & (s_e >= P)
            wgt = (first & ~match).astype(jnp.float32)
            relu_e = jnp.maximum(rows_a + rows_b, 0.0) * wgt[:, None]
            aggu = jax.ops.segment_sum(relu_e, d_e, num_segments=P)
            cnt = jax.ops.segment_sum(wgt, d_e, num_segments=P)
            return aggu, cnt

        aggu, cnt = lax.cond(m_rel <= cap,
                             lambda: base_agg(ks[:cap], True),
                             lambda: base_agg(ks, False))
        cnt = cnt + 10.0

        relu_n = jnp.maximum(a_tab[nn + P, :64] + bb[:, None, :], 0.0)
        aggu = aggu + jnp.sum(relu_n, axis=1)

        agg = aggu @ l0b_w + cnt[:, None] * l0b_b
        hnp = xp + jnp.maximum(agg @ l1a_w + l1a_b, 0.0) @ l1b_w + l1b_b
        sm = hnp @ sm_w + sm_b
        path = path.at[1:-1].set(sm[1:-1])

    return path


# speedup vs baseline: 5.7897x; 5.7885x over previous
import functools
import jax
import jax.numpy as jnp
from jax import lax
from jax.experimental import pallas as pl
from jax.experimental.pallas import tpu as pltpu

_TR = 2000


import numpy as np

_I0 = np.int32(0)


def _stage1_body(xn_ref, w1_ref, b1_ref, h_ref, s1_ref, s2_ref):
    h = jnp.dot(xn_ref[...], w1_ref[...], preferred_element_type=jnp.float32)
    h = h + b1_ref[...]
    h_ref[...] = h
    s1_ref[...] = jnp.sum(h, axis=0, keepdims=True)[None]
    s2_ref[...] = jnp.sum(h * h, axis=0, keepdims=True)[None]


def _node_stage1(xn, w1, b1):
    n = xn.shape[0]
    g = n // _TR
    kf = pl.pallas_call(
        _stage1_body,
        grid=(g,),
        in_specs=[
            pl.BlockSpec((_TR, xn.shape[1]), lambda i: (i, _I0)),
            pl.BlockSpec((xn.shape[1], 64), lambda i: (_I0, _I0)),
            pl.BlockSpec((1, 64), lambda i: (_I0, _I0)),
        ],
        out_specs=[
            pl.BlockSpec((_TR, 64), lambda i: (i, _I0)),
            pl.BlockSpec((1, 1, 64), lambda i: (i, _I0, _I0)),
            pl.BlockSpec((1, 1, 64), lambda i: (i, _I0, _I0)),
        ],
        out_shape=[
            jax.ShapeDtypeStruct((n, 64), jnp.float32),
            jax.ShapeDtypeStruct((g, 1, 64), jnp.float32),
            jax.ShapeDtypeStruct((g, 1, 64), jnp.float32),
        ],
    )
    return kf(xn, w1, b1.reshape(1, 64))


def _stage2_body(h_ref, sc_ref, tt_ref, w2_ref, b2_ref, wa_ref, x_ref, a_ref):
    hb = h_ref[...] * sc_ref[...] + tt_ref[...]
    x = jnp.dot(jnp.maximum(hb, 0.0), w2_ref[...],
                preferred_element_type=jnp.float32) + b2_ref[...]
    x_ref[...] = x
    a_ref[...] = jnp.dot(x, wa_ref[...], preferred_element_type=jnp.float32)


def _node_stage2(h, sc, tt, w2, b2, wa):
    n = h.shape[0]
    g = n // _TR
    kf = pl.pallas_call(
        _stage2_body,
        grid=(g,),
        in_specs=[
            pl.BlockSpec((_TR, 64), lambda i: (i, _I0)),
            pl.BlockSpec((1, 64), lambda i: (_I0, _I0)),
            pl.BlockSpec((1, 64), lambda i: (_I0, _I0)),
            pl.BlockSpec((64, 64), lambda i: (_I0, _I0)),
            pl.BlockSpec((1, 64), lambda i: (_I0, _I0)),
            pl.BlockSpec((64, 64), lambda i: (_I0, _I0)),
        ],
        out_specs=[
            pl.BlockSpec((_TR, 64), lambda i: (i, _I0)),
            pl.BlockSpec((_TR, 64), lambda i: (i, _I0)),
        ],
        out_shape=[
            jax.ShapeDtypeStruct((n, 64), jnp.float32),
            jax.ShapeDtypeStruct((n, 64), jnp.float32),
        ],
    )
    return kf(h, sc.reshape(1, 64), tt.reshape(1, 64), w2, b2.reshape(1, 64), wa)


def kernel(path, free, collided, obstacles, edge_index, loop, node_w1, node_b1,
           bn_g, bn_b, node_w2, node_b2, l0a_w, l0a_b, l0b_w, l0b_b,
           l1a_w, l1a_b, l1b_w, l1b_b, sm_w, sm_b):
    P = path.shape[0]
    Fn = free.shape[0]
    C = collided.shape[0]
    N = P + Fn + C

    path = path.astype(jnp.float32)
    cand = jnp.concatenate([free, collided], axis=0).astype(jnp.float32)

    src0 = edge_index[0]
    dst0 = edge_index[1]
    sent64 = jnp.int64(2**31 - 1)
    keys = jnp.where(dst0 < P, dst0.astype(jnp.int64) * N + src0.astype(jnp.int64), sent64)
    ks = jnp.sort(keys.astype(jnp.int32))
    sent = jnp.int32(2**31 - 1)
    m_rel = jnp.sum((ks < sent).astype(jnp.int32))
    cap = 65536

    r = jnp.arange(N, dtype=jnp.int32)
    info = jnp.stack([(r < P).astype(jnp.float32),
                      ((r >= P) & (r < P + Fn)).astype(jnp.float32),
                      (r >= P + Fn).astype(jnp.float32)], axis=1)

    w0 = l0a_w[:64]
    w1m = l0a_w[64:128]
    w2m = l0a_w[128:]
    wa = w0 + w1m
    wb = w2m - w0

    c2 = jnp.sum(cand * cand, axis=1)

    for _ in range(2):
        nodes = jnp.concatenate([path, cand], axis=0)
        xn = jnp.concatenate([nodes, info], axis=1)
        h, s1, s2 = _node_stage1(xn, node_w1, node_b1)
        mu = jnp.sum(s1, axis=(0, 1)) / N
        var = jnp.sum(s2, axis=(0, 1)) / N - mu * mu
        sc = bn_g / jnp.sqrt(var + 1e-5)
        tt = bn_b - mu * sc
        x, a_tab = _node_stage2(h, sc, tt, node_w2, node_b2, wa)
        xp = x[:P]
        bb = xp @ wb + l0a_b

        p2 = jnp.sum(path * path, axis=1)
        d2 = p2[:, None] + c2[None, :] - 2.0 * (path @ cand.T)
        nb2 = 25
        w2b = d2.shape[1] // nb2
        bv, bi = lax.top_k(-d2.reshape(P, nb2, w2b), 10)
        gi = bi + (jnp.arange(nb2, dtype=jnp.int32) * w2b)[None, :, None]
        fv, fi = lax.top_k(bv.reshape(P, nb2 * 10), 10)
        nn = jnp.take_along_axis(gi.reshape(P, nb2 * 10), fi, axis=1)

        def base_agg(ks_slice):
            valid = ks_slice < sent
            first = jnp.concatenate(
                [valid[:1], (ks_slice[1:] != ks_slice[:-1]) & valid[1:]])
            d_e = jnp.where(valid, ks_slice // N, 0)
            s_e = jnp.where(valid, ks_slice % N, 0)
            nnrows = nn[d_e]
            match = jnp.any(nnrows == (s_e - P)[:, None], axis=1) & (s_e >= P)
            wgt = (first & ~match).astype(jnp.float32)
            relu_e = jnp.maximum(a_tab[s_e] + bb[d_e], 0.0) * wgt[:, None]
            aggu = jax.ops.segment_sum(relu_e, d_e, num_segments=P)
            cnt = jax.ops.segment_sum(wgt, d_e, num_segments=P)
            return aggu, cnt

        aggu, cnt = lax.cond(m_rel <= cap,
                             lambda: base_agg(ks[:cap]),
                             lambda: base_agg(ks))
        cnt = cnt + 10.0

        relu_n = jnp.maximum(a_tab[nn + P] + bb[:, None, :], 0.0)
        aggu = aggu + jnp.sum(relu_n, axis=1)

        agg = aggu @ l0b_w + cnt[:, None] * l0b_b
        hnp = xp + jnp.maximum(agg @ l1a_w + l1a_b, 0.0) @ l1b_w + l1b_b
        sm = hnp @ sm_w + sm_b
        path = path.at[1:-1].set(sm[1:-1])

    return path


# speedup vs baseline: 15.9534x; 2.7555x over previous
import functools
import jax
import jax.numpy as jnp
from jax import lax
from jax.experimental import pallas as pl
from jax.experimental.pallas import tpu as pltpu

_TR = 2000


import numpy as np

_I0 = np.int32(0)


def _stage1_body(xn_ref, w1_ref, b1_ref, h_ref, s1_ref, s2_ref):
    h = jnp.dot(xn_ref[...], w1_ref[...], preferred_element_type=jnp.float32)
    h = h + b1_ref[...]
    h_ref[...] = h
    s1_ref[...] = jnp.sum(h, axis=0, keepdims=True)[None]
    s2_ref[...] = jnp.sum(h * h, axis=0, keepdims=True)[None]


def _node_stage1(xn, w1, b1):
    n = xn.shape[0]
    g = n // _TR
    kf = pl.pallas_call(
        _stage1_body,
        grid=(g,),
        in_specs=[
            pl.BlockSpec((_TR, xn.shape[1]), lambda i: (i, _I0)),
            pl.BlockSpec((xn.shape[1], 64), lambda i: (_I0, _I0)),
            pl.BlockSpec((1, 64), lambda i: (_I0, _I0)),
        ],
        out_specs=[
            pl.BlockSpec((_TR, 64), lambda i: (i, _I0)),
            pl.BlockSpec((1, 1, 64), lambda i: (i, _I0, _I0)),
            pl.BlockSpec((1, 1, 64), lambda i: (i, _I0, _I0)),
        ],
        out_shape=[
            jax.ShapeDtypeStruct((n, 64), jnp.float32),
            jax.ShapeDtypeStruct((g, 1, 64), jnp.float32),
            jax.ShapeDtypeStruct((g, 1, 64), jnp.float32),
        ],
    )
    return kf(xn, w1, b1.reshape(1, 64))


def _stage2_body(h_ref, sc_ref, tt_ref, w2_ref, b2_ref, wa_ref, x_ref, a_ref):
    hb = h_ref[...] * sc_ref[...] + tt_ref[...]
    x = jnp.dot(jnp.maximum(hb, 0.0), w2_ref[...],
                preferred_element_type=jnp.float32) + b2_ref[...]
    x_ref[...] = x
    a_ref[...] = jnp.dot(x, wa_ref[...], preferred_element_type=jnp.float32)


def _node_stage2(h, sc, tt, w2, b2, wa):
    n = h.shape[0]
    g = n // _TR
    kf = pl.pallas_call(
        _stage2_body,
        grid=(g,),
        in_specs=[
            pl.BlockSpec((_TR, 64), lambda i: (i, _I0)),
            pl.BlockSpec((1, 64), lambda i: (_I0, _I0)),
            pl.BlockSpec((1, 64), lambda i: (_I0, _I0)),
            pl.BlockSpec((64, 64), lambda i: (_I0, _I0)),
            pl.BlockSpec((1, 64), lambda i: (_I0, _I0)),
            pl.BlockSpec((64, 64), lambda i: (_I0, _I0)),
        ],
        out_specs=[
            pl.BlockSpec((_TR, 64), lambda i: (i, _I0)),
            pl.BlockSpec((_TR, 64), lambda i: (i, _I0)),
        ],
        out_shape=[
            jax.ShapeDtypeStruct((n, 64), jnp.float32),
            jax.ShapeDtypeStruct((n, 64), jnp.float32),
        ],
    )
    return kf(h, sc.reshape(1, 64), tt.reshape(1, 64), w2, b2.reshape(1, 64), wa)


def kernel(path, free, collided, obstacles, edge_index, loop, node_w1, node_b1,
           bn_g, bn_b, node_w2, node_b2, l0a_w, l0a_b, l0b_w, l0b_b,
           l1a_w, l1a_b, l1b_w, l1b_b, sm_w, sm_b):
    P = path.shape[0]
    Fn = free.shape[0]
    C = collided.shape[0]
    N = P + Fn + C

    path = path.astype(jnp.float32)
    cand = jnp.concatenate([free, collided], axis=0).astype(jnp.float32)

    src0 = edge_index[0]
    dst0 = edge_index[1]
    sent64 = jnp.int64(2**31 - 1)
    keys = jnp.where(dst0 < P, dst0.astype(jnp.int64) * N + src0.astype(jnp.int64), sent64)
    ks = jnp.sort(keys.astype(jnp.int32))
    sent = jnp.int32(2**31 - 1)
    m_rel = jnp.sum((ks < sent).astype(jnp.int32))
    cap = 65536

    r = jnp.arange(N, dtype=jnp.int32)
    info = jnp.stack([(r < P).astype(jnp.float32),
                      ((r >= P) & (r < P + Fn)).astype(jnp.float32),
                      (r >= P + Fn).astype(jnp.float32)], axis=1)

    w0 = l0a_w[:64]
    w1m = l0a_w[64:128]
    w2m = l0a_w[128:]
    wa = w0 + w1m
    wb = w2m - w0

    c2 = jnp.sum(cand * cand, axis=1)

    for _ in range(2):
        nodes = jnp.concatenate([path, cand], axis=0)
        xn = jnp.concatenate([nodes, info], axis=1)
        h, s1, s2 = _node_stage1(xn, node_w1, node_b1)
        mu = jnp.sum(s1, axis=(0, 1)) / N
        var = jnp.sum(s2, axis=(0, 1)) / N - mu * mu
        sc = bn_g / jnp.sqrt(var + 1e-5)
        tt = bn_b - mu * sc
        x, a_tab = _node_stage2(h, sc, tt, node_w2, node_b2, wa)
        xp = x[:P]
        bb = xp @ wb + l0a_b

        p2 = jnp.sum(path * path, axis=1)
        d2 = p2[:, None] + c2[None, :] - 2.0 * (path @ cand.T)
        _, nn = lax.top_k(-d2, 10)

        def base_agg(ks_slice):
            valid = ks_slice < sent
            first = jnp.concatenate(
                [valid[:1], (ks_slice[1:] != ks_slice[:-1]) & valid[1:]])
            d_e = jnp.where(valid, ks_slice // N, 0)
            s_e = jnp.where(valid, ks_slice % N, 0)
            nnrows = nn[d_e]
            match = jnp.any(nnrows == (s_e - P)[:, None], axis=1) & (s_e >= P)
            wgt = (first & ~match).astype(jnp.float32)
            relu_e = jnp.maximum(a_tab[s_e] + bb[d_e], 0.0) * wgt[:, None]
            aggu = jax.ops.segment_sum(relu_e, d_e, num_segments=P)
            cnt = jax.ops.segment_sum(wgt, d_e, num_segments=P)
            return aggu, cnt

        aggu, cnt = lax.cond(m_rel <= cap,
                             lambda: base_agg(ks[:cap]),
                             lambda: base_agg(ks))
        cnt = cnt + 10.0

        relu_n = jnp.maximum(a_tab[nn + P] + bb[:, None, :], 0.0)
        aggu = aggu + jnp.sum(relu_n, axis=1)

        agg = aggu @ l0b_w + cnt[:, None] * l0b_b
        hnp = xp + jnp.maximum(agg @ l1a_w + l1a_b, 0.0) @ l1b_w + l1b_b
        sm = hnp @ sm_w + sm_b
        path = path.at[1:-1].set(sm[1:-1])

    return path


# speedup vs baseline: 28.1465x; 1.7643x over previous
import functools
import jax
import jax.numpy as jnp
from jax import lax
from jax.experimental import pallas as pl
from jax.experimental.pallas import tpu as pltpu

_TR = 2000


import numpy as np

_I0 = np.int32(0)


def _stage1_body(xn_ref, w1_ref, b1_ref, h_ref, s1_ref, s2_ref):
    h = jnp.dot(xn_ref[...], w1_ref[...], preferred_element_type=jnp.float32)
    h = h + b1_ref[...]
    h_ref[...] = h
    s1_ref[...] = jnp.sum(h, axis=0, keepdims=True)[None]
    s2_ref[...] = jnp.sum(h * h, axis=0, keepdims=True)[None]


def _node_stage1(xn, w1, b1):
    n = xn.shape[0]
    g = n // _TR
    kf = pl.pallas_call(
        _stage1_body,
        grid=(g,),
        in_specs=[
            pl.BlockSpec((_TR, xn.shape[1]), lambda i: (i, _I0)),
            pl.BlockSpec((xn.shape[1], 64), lambda i: (_I0, _I0)),
            pl.BlockSpec((1, 64), lambda i: (_I0, _I0)),
        ],
        out_specs=[
            pl.BlockSpec((_TR, 64), lambda i: (i, _I0)),
            pl.BlockSpec((1, 1, 64), lambda i: (i, _I0, _I0)),
            pl.BlockSpec((1, 1, 64), lambda i: (i, _I0, _I0)),
        ],
        out_shape=[
            jax.ShapeDtypeStruct((n, 64), jnp.float32),
            jax.ShapeDtypeStruct((g, 1, 64), jnp.float32),
            jax.ShapeDtypeStruct((g, 1, 64), jnp.float32),
        ],
    )
    return kf(xn, w1, b1.reshape(1, 64))


def _stage2_body(h_ref, sc_ref, tt_ref, w2_ref, b2_ref, wa_ref, x_ref, a_ref):
    hb = h_ref[...] * sc_ref[...] + tt_ref[...]
    x = jnp.dot(jnp.maximum(hb, 0.0), w2_ref[...],
                preferred_element_type=jnp.float32) + b2_ref[...]
    x_ref[...] = x
    a_ref[...] = jnp.dot(x, wa_ref[...], preferred_element_type=jnp.float32)


def _node_stage2(h, sc, tt, w2, b2, wa):
    n = h.shape[0]
    g = n // _TR
    kf = pl.pallas_call(
        _stage2_body,
        grid=(g,),
        in_specs=[
            pl.BlockSpec((_TR, 64), lambda i: (i, _I0)),
            pl.BlockSpec((1, 64), lambda i: (_I0, _I0)),
            pl.BlockSpec((1, 64), lambda i: (_I0, _I0)),
            pl.BlockSpec((64, 64), lambda i: (_I0, _I0)),
            pl.BlockSpec((1, 64), lambda i: (_I0, _I0)),
            pl.BlockSpec((64, 64), lambda i: (_I0, _I0)),
        ],
        out_specs=[
            pl.BlockSpec((_TR, 64), lambda i: (i, _I0)),
            pl.BlockSpec((_TR, 64), lambda i: (i, _I0)),
        ],
        out_shape=[
            jax.ShapeDtypeStruct((n, 64), jnp.float32),
            jax.ShapeDtypeStruct((n, 64), jnp.float32),
        ],
    )
    return kf(h, sc.reshape(1, 64), tt.reshape(1, 64), w2, b2.reshape(1, 64), wa)


def kernel(path, free, collided, obstacles, edge_index, loop, node_w1, node_b1,
           bn_g, bn_b, node_w2, node_b2, l0a_w, l0a_b, l0b_w, l0b_b,
           l1a_w, l1a_b, l1b_w, l1b_b, sm_w, sm_b):
    P = path.shape[0]
    Fn = free.shape[0]
    C = collided.shape[0]
    N = P + Fn + C

    path = path.astype(jnp.float32)
    cand = jnp.concatenate([free, collided], axis=0).astype(jnp.float32)

    src0 = edge_index[0]
    dst0 = edge_index[1]
    sent64 = jnp.int64(2**31 - 1)
    keys = jnp.where(dst0 < P, dst0.astype(jnp.int64) * N + src0.astype(jnp.int64), sent64)
    ks = jnp.sort(keys.astype(jnp.int32))
    sent = jnp.int32(2**31 - 1)
    m_rel = jnp.sum((ks < sent).astype(jnp.int32))
    cap = 20480

    r = jnp.arange(N, dtype=jnp.int32)
    info = jnp.stack([(r < P).astype(jnp.float32),
                      ((r >= P) & (r < P + Fn)).astype(jnp.float32),
                      (r >= P + Fn).astype(jnp.float32)], axis=1)

    w0 = l0a_w[:64]
    w1m = l0a_w[64:128]
    w2m = l0a_w[128:]
    wa = w0 + w1m
    wb = w2m - w0

    c2 = jnp.sum(cand * cand, axis=1)

    for _ in range(2):
        nodes = jnp.concatenate([path, cand], axis=0)
        xn = jnp.concatenate([nodes, info], axis=1)
        h, s1, s2 = _node_stage1(xn, node_w1, node_b1)
        mu = jnp.sum(s1, axis=(0, 1)) / N
        var = jnp.sum(s2, axis=(0, 1)) / N - mu * mu
        sc = bn_g / jnp.sqrt(var + 1e-5)
        tt = bn_b - mu * sc
        x, a_tab = _node_stage2(h, sc, tt, node_w2, node_b2, wa)
        xp = x[:P]
        bb = xp @ wb + l0a_b

        p2 = jnp.sum(path * path, axis=1)
        d2 = p2[:, None] + c2[None, :] - 2.0 * (path @ cand.T)
        _, nn = lax.top_k(-d2, 10)

        def base_agg(ks_slice):
            valid = ks_slice < sent
            first = jnp.concatenate(
                [valid[:1], (ks_slice[1:] != ks_slice[:-1]) & valid[1:]])
            d_e = jnp.where(valid, ks_slice // N, 0)
            s_e = jnp.where(valid, ks_slice % N, 0)
            nnrows = nn[d_e]
            match = jnp.any(nnrows == (s_e - P)[:, None], axis=1) & (s_e >= P)
            wgt = (first & ~match).astype(jnp.float32)
            relu_e = jnp.maximum(a_tab[s_e] + bb[d_e], 0.0) * wgt[:, None]
            aggu = jax.ops.segment_sum(relu_e, d_e, num_segments=P)
            cnt = jax.ops.segment_sum(wgt, d_e, num_segments=P)
            return aggu, cnt

        aggu, cnt = lax.cond(m_rel <= cap,
                             lambda: base_agg(ks[:cap]),
                             lambda: base_agg(ks))
        cnt = cnt + 10.0

        relu_n = jnp.maximum(a_tab[nn + P] + bb[:, None, :], 0.0)
        aggu = aggu + jnp.sum(relu_n, axis=1)

        agg = aggu @ l0b_w + cnt[:, None] * l0b_b
        hnp = xp + jnp.maximum(agg @ l1a_w + l1a_b, 0.0) @ l1b_w + l1b_b
        sm = hnp @ sm_w + sm_b
        path = path.at[1:-1].set(sm[1:-1])

    return path


# speedup vs baseline: 28.1922x; 1.0016x over previous
import functools
import jax
import jax.numpy as jnp
from jax import lax
from jax.experimental import pallas as pl
from jax.experimental.pallas import tpu as pltpu

_TR = 2000


import numpy as np

_I0 = np.int32(0)


def _stage1_body(xn_ref, w1_ref, b1_ref, h_ref, s1_ref, s2_ref):
    h = jnp.dot(xn_ref[...], w1_ref[...], preferred_element_type=jnp.float32)
    h = h + b1_ref[...]
    h_ref[...] = h
    s1_ref[...] = jnp.sum(h, axis=0, keepdims=True)[None]
    s2_ref[...] = jnp.sum(h * h, axis=0, keepdims=True)[None]


def _node_stage1(xn, w1, b1):
    n = xn.shape[0]
    g = n // _TR
    kf = pl.pallas_call(
        _stage1_body,
        grid=(g,),
        in_specs=[
            pl.BlockSpec((_TR, xn.shape[1]), lambda i: (i, _I0)),
            pl.BlockSpec((xn.shape[1], 64), lambda i: (_I0, _I0)),
            pl.BlockSpec((1, 64), lambda i: (_I0, _I0)),
        ],
        out_specs=[
            pl.BlockSpec((_TR, 64), lambda i: (i, _I0)),
            pl.BlockSpec((1, 1, 64), lambda i: (i, _I0, _I0)),
            pl.BlockSpec((1, 1, 64), lambda i: (i, _I0, _I0)),
        ],
        out_shape=[
            jax.ShapeDtypeStruct((n, 64), jnp.float32),
            jax.ShapeDtypeStruct((g, 1, 64), jnp.float32),
            jax.ShapeDtypeStruct((g, 1, 64), jnp.float32),
        ],
    )
    return kf(xn, w1, b1.reshape(1, 64))


def _stage2_body(h_ref, sc_ref, tt_ref, w2_ref, b2_ref, wa_ref, x_ref, a_ref):
    hb = h_ref[...] * sc_ref[...] + tt_ref[...]
    x = jnp.dot(jnp.maximum(hb, 0.0), w2_ref[...],
                preferred_element_type=jnp.float32) + b2_ref[...]
    x_ref[...] = x
    a_ref[...] = jnp.dot(x, wa_ref[...], preferred_element_type=jnp.float32)


def _node_stage2(h, sc, tt, w2, b2, wa):
    n = h.shape[0]
    g = n // _TR
    kf = pl.pallas_call(
        _stage2_body,
        grid=(g,),
        in_specs=[
            pl.BlockSpec((_TR, 64), lambda i: (i, _I0)),
            pl.BlockSpec((1, 64), lambda i: (_I0, _I0)),
            pl.BlockSpec((1, 64), lambda i: (_I0, _I0)),
            pl.BlockSpec((64, 64), lambda i: (_I0, _I0)),
            pl.BlockSpec((1, 64), lambda i: (_I0, _I0)),
            pl.BlockSpec((64, 64), lambda i: (_I0, _I0)),
        ],
        out_specs=[
            pl.BlockSpec((_TR, 64), lambda i: (i, _I0)),
            pl.BlockSpec((_TR, 64), lambda i: (i, _I0)),
        ],
        out_shape=[
            jax.ShapeDtypeStruct((n, 64), jnp.float32),
            jax.ShapeDtypeStruct((n, 64), jnp.float32),
        ],
    )
    return kf(h, sc.reshape(1, 64), tt.reshape(1, 64), w2, b2.reshape(1, 64), wa)


def kernel(path, free, collided, obstacles, edge_index, loop, node_w1, node_b1,
           bn_g, bn_b, node_w2, node_b2, l0a_w, l0a_b, l0b_w, l0b_b,
           l1a_w, l1a_b, l1b_w, l1b_b, sm_w, sm_b):
    P = path.shape[0]
    Fn = free.shape[0]
    C = collided.shape[0]
    N = P + Fn + C

    path = path.astype(jnp.float32)
    cand = jnp.concatenate([free, collided], axis=0).astype(jnp.float32)

    src0 = edge_index[0]
    dst0 = edge_index[1]
    sent64 = jnp.int64(2**31 - 1)
    keys = jnp.where(dst0 < P, dst0.astype(jnp.int64) * N + src0.astype(jnp.int64), sent64)
    ks = jnp.sort(keys.astype(jnp.int32))
    sent = jnp.int32(2**31 - 1)
    m_rel = jnp.sum((ks < sent).astype(jnp.int32))
    cap = 20480

    r = jnp.arange(N, dtype=jnp.int32)
    info = jnp.stack([(r < P).astype(jnp.float32),
                      ((r >= P) & (r < P + Fn)).astype(jnp.float32),
                      (r >= P + Fn).astype(jnp.float32)], axis=1)

    w0 = l0a_w[:64]
    w1m = l0a_w[64:128]
    w2m = l0a_w[128:]
    wa = w0 + w1m
    wb = w2m - w0

    c2 = jnp.sum(cand * cand, axis=1)

    for _ in range(2):
        nodes = jnp.concatenate([path, cand], axis=0)
        xn = jnp.concatenate([nodes, info], axis=1)
        h, s1, s2 = _node_stage1(xn, node_w1, node_b1)
        mu = jnp.sum(s1, axis=(0, 1)) / N
        var = jnp.sum(s2, axis=(0, 1)) / N - mu * mu
        sc = bn_g / jnp.sqrt(var + 1e-5)
        tt = bn_b - mu * sc
        x, a_tab = _node_stage2(h, sc, tt, node_w2, node_b2, wa)
        xp = x[:P]
        bb = xp @ wb + l0a_b

        p2 = jnp.sum(path * path, axis=1)
        d2n = 2.0 * (path @ cand.T) - p2[:, None] - c2[None, :]
        _, nn = lax.approx_max_k(d2n, 10, recall_target=1.0)

        def base_agg(ks_slice):
            valid = ks_slice < sent
            first = jnp.concatenate(
                [valid[:1], (ks_slice[1:] != ks_slice[:-1]) & valid[1:]])
            d_e = jnp.where(valid, ks_slice // N, 0)
            s_e = jnp.where(valid, ks_slice % N, 0)
            nnrows = nn[d_e]
            match = jnp.any(nnrows == (s_e - P)[:, None], axis=1) & (s_e >= P)
            wgt = (first & ~match).astype(jnp.float32)
            relu_e = jnp.maximum(a_tab[s_e] + bb[d_e], 0.0) * wgt[:, None]
            aggu = jax.ops.segment_sum(relu_e, d_e, num_segments=P)
            cnt = jax.ops.segment_sum(wgt, d_e, num_segments=P)
            return aggu, cnt

        aggu, cnt = lax.cond(m_rel <= cap,
                             lambda: base_agg(ks[:cap]),
                             lambda: base_agg(ks))
        cnt = cnt + 10.0

        relu_n = jnp.maximum(a_tab[nn + P] + bb[:, None, :], 0.0)
        aggu = aggu + jnp.sum(relu_n, axis=1)

        agg = aggu @ l0b_w + cnt[:, None] * l0b_b
        hnp = xp + jnp.maximum(agg @ l1a_w + l1a_b, 0.0) @ l1b_w + l1b_b
        sm = hnp @ sm_w + sm_b
        path = path.at[1:-1].set(sm[1:-1])

    return path


# speedup vs baseline: 29.0152x; 1.0292x over previous
import functools
import jax
import jax.numpy as jnp
from jax import lax
from jax.experimental import pallas as pl
from jax.experimental.pallas import tpu as pltpu

_TR = 2000


import numpy as np

_I0 = np.int32(0)


def _stage1_body(xn_ref, w1_ref, b1_ref, h_ref, s1_ref, s2_ref):
    h = jnp.dot(xn_ref[...], w1_ref[...], preferred_element_type=jnp.float32)
    h = h + b1_ref[...]
    h_ref[...] = h
    s1_ref[...] = jnp.sum(h, axis=0, keepdims=True)[None]
    s2_ref[...] = jnp.sum(h * h, axis=0, keepdims=True)[None]


def _node_stage1(xn, w1, b1):
    n = xn.shape[0]
    g = n // _TR
    kf = pl.pallas_call(
        _stage1_body,
        grid=(g,),
        in_specs=[
            pl.BlockSpec((_TR, xn.shape[1]), lambda i: (i, _I0)),
            pl.BlockSpec((xn.shape[1], 64), lambda i: (_I0, _I0)),
            pl.BlockSpec((1, 64), lambda i: (_I0, _I0)),
        ],
        out_specs=[
            pl.BlockSpec((_TR, 64), lambda i: (i, _I0)),
            pl.BlockSpec((1, 1, 64), lambda i: (i, _I0, _I0)),
            pl.BlockSpec((1, 1, 64), lambda i: (i, _I0, _I0)),
        ],
        out_shape=[
            jax.ShapeDtypeStruct((n, 64), jnp.float32),
            jax.ShapeDtypeStruct((g, 1, 64), jnp.float32),
            jax.ShapeDtypeStruct((g, 1, 64), jnp.float32),
        ],
    )
    return kf(xn, w1, b1.reshape(1, 64))


def _stage2_body(h_ref, sc_ref, tt_ref, w2_ref, b2_ref, wa_ref, x_ref, a_ref):
    hb = h_ref[...] * sc_ref[...] + tt_ref[...]
    x = jnp.dot(jnp.maximum(hb, 0.0), w2_ref[...],
                preferred_element_type=jnp.float32) + b2_ref[...]
    x_ref[...] = x
    a_ref[...] = jnp.dot(x, wa_ref[...], preferred_element_type=jnp.float32)


def _node_stage2(h, sc, tt, w2, b2, wa):
    n = h.shape[0]
    g = n // _TR
    kf = pl.pallas_call(
        _stage2_body,
        grid=(g,),
        in_specs=[
            pl.BlockSpec((_TR, 64), lambda i: (i, _I0)),
            pl.BlockSpec((1, 64), lambda i: (_I0, _I0)),
            pl.BlockSpec((1, 64), lambda i: (_I0, _I0)),
            pl.BlockSpec((64, 64), lambda i: (_I0, _I0)),
            pl.BlockSpec((1, 64), lambda i: (_I0, _I0)),
            pl.BlockSpec((64, 64), lambda i: (_I0, _I0)),
        ],
        out_specs=[
            pl.BlockSpec((_TR, 64), lambda i: (i, _I0)),
            pl.BlockSpec((_TR, 64), lambda i: (i, _I0)),
        ],
        out_shape=[
            jax.ShapeDtypeStruct((n, 64), jnp.float32),
            jax.ShapeDtypeStruct((n, 64), jnp.float32),
        ],
    )
    return kf(h, sc.reshape(1, 64), tt.reshape(1, 64), w2, b2.reshape(1, 64), wa)


def kernel(path, free, collided, obstacles, edge_index, loop, node_w1, node_b1,
           bn_g, bn_b, node_w2, node_b2, l0a_w, l0a_b, l0b_w, l0b_b,
           l1a_w, l1a_b, l1b_w, l1b_b, sm_w, sm_b):
    P = path.shape[0]
    Fn = free.shape[0]
    C = collided.shape[0]
    N = P + Fn + C

    path = path.astype(jnp.float32)
    cand = jnp.concatenate([free, collided], axis=0).astype(jnp.float32)

    src0 = edge_index[0]
    dst0 = edge_index[1]
    sent64 = jnp.int64(2**31 - 1)
    keys = jnp.where(dst0 < P, dst0.astype(jnp.int64) * N + src0.astype(jnp.int64), sent64)
    ks = jnp.sort(keys.astype(jnp.int32))
    sent = jnp.int32(2**31 - 1)
    m_rel = jnp.sum((ks < sent).astype(jnp.int32))
    cap = 18432

    r = jnp.arange(N, dtype=jnp.int32)
    info = jnp.stack([(r < P).astype(jnp.float32),
                      ((r >= P) & (r < P + Fn)).astype(jnp.float32),
                      (r >= P + Fn).astype(jnp.float32)], axis=1)

    w0 = l0a_w[:64]
    w1m = l0a_w[64:128]
    w2m = l0a_w[128:]
    wa = w0 + w1m
    wb = w2m - w0

    c2 = jnp.sum(cand * cand, axis=1)

    for _ in range(2):
        nodes = jnp.concatenate([path, cand], axis=0)
        xn = jnp.concatenate([nodes, info], axis=1)
        h, s1, s2 = _node_stage1(xn, node_w1, node_b1)
        mu = jnp.sum(s1, axis=(0, 1)) / N
        var = jnp.sum(s2, axis=(0, 1)) / N - mu * mu
        sc = bn_g / jnp.sqrt(var + 1e-5)
        tt = bn_b - mu * sc
        x, a_tab = _node_stage2(h, sc, tt, node_w2, node_b2, wa)
        xp = x[:P]
        bb = xp @ wb + l0a_b

        p2 = jnp.sum(path * path, axis=1)
        d2 = p2[:, None] + c2[None, :] - 2.0 * (path @ cand.T)
        _, nn = lax.top_k(-d2, 10)

        def base_agg(ks_slice):
            valid = ks_slice < sent
            first = jnp.concatenate(
                [valid[:1], (ks_slice[1:] != ks_slice[:-1]) & valid[1:]])
            d_e = jnp.where(valid, ks_slice // N, 0)
            s_e = jnp.where(valid, ks_slice % N, 0)
            nnrows = nn[d_e]
            match = jnp.any(nnrows == (s_e - P)[:, None], axis=1) & (s_e >= P)
            wgt = (first & ~match).astype(jnp.float32)
            relu_e = jnp.maximum(a_tab[s_e] + bb[d_e], 0.0) * wgt[:, None]
            aggu = jax.ops.segment_sum(relu_e, d_e, num_segments=P)
            cnt = jax.ops.segment_sum(wgt, d_e, num_segments=P)
            return aggu, cnt

        aggu, cnt = lax.cond(m_rel <= cap,
                             lambda: base_agg(ks[:cap]),
                             lambda: base_agg(ks))
        cnt = cnt + 10.0

        relu_n = jnp.maximum(a_tab[nn + P] + bb[:, None, :], 0.0)
        aggu = aggu + jnp.sum(relu_n, axis=1)

        agg = aggu @ l0b_w + cnt[:, None] * l0b_b
        hnp = xp + jnp.maximum(agg @ l1a_w + l1a_b, 0.0) @ l1b_w + l1b_b
        sm = hnp @ sm_w + sm_b
        path = path.at[1:-1].set(sm[1:-1])

    return path
